# Initial kernel scaffold; baseline (speedup 1.0000x reference)
import functools

import jax
import jax.numpy as jnp
from jax import lax
from jax.experimental import pallas as pl
from jax.experimental.pallas import tpu as pltpu
from jax.experimental.pallas import tpu_sc as plsc

NUM_CLASSES = 1000
FEAT = 2048
BATCH = 16384
SIGMA = 0.2

NC = 2
NS = 16
ROWS_PER_TILE = BATCH // NS
CHUNK = 16
N_CHUNKS = ROWS_PER_TILE // CHUNK
DRAIN = 8
N_DRAIN = NUM_CLASSES // DRAIN
DRAIN_ITERS = (N_DRAIN + NS - 1) // NS


def _sc_segment_sums(rgb_feats, ir_feats, rgb_labels, ir_labels, zf, zc):
  f32 = jnp.float32
  mesh = plsc.VectorSubcoreMesh(
      core_axis_name="c", subcore_axis_name="s",
      num_cores=NC, num_subcores=NS)
  out_type = (
      jax.ShapeDtypeStruct((NUM_CLASSES, FEAT), f32),
      jax.ShapeDtypeStruct((NUM_CLASSES, FEAT), f32),
      jax.ShapeDtypeStruct((NUM_CLASSES, CHUNK), f32),
      jax.ShapeDtypeStruct((NUM_CLASSES, CHUNK), f32),
  )
  scratch = (
      pltpu.VMEM_SHARED((NUM_CLASSES, FEAT), f32),
      pltpu.VMEM_SHARED((NUM_CLASSES, CHUNK), f32),
      pltpu.VMEM((CHUNK,), jnp.int32),
      pltpu.VMEM((CHUNK, FEAT), f32),
      pltpu.VMEM((CHUNK, CHUNK), f32),
  )

  @functools.partial(pl.kernel, out_type=out_type, mesh=mesh,
                     scratch_types=scratch)
  def body(rgb_hbm, ir_hbm, rgb_lab_hbm, ir_lab_hbm, zf_hbm, zc_hbm,
           vis_sum_hbm, ir_sum_hbm, vis_cnt_hbm, ir_cnt_hbm,
           acc_sh, cnt_sh, idx_v, chunk_v, ones_v):
    cid = lax.axis_index("c")
    sid = lax.axis_index("s")

    def zero_step(i, carry):
      k = sid + NS * i

      @pl.when(k < N_DRAIN)
      def _():
        sl = pl.ds(k * DRAIN, DRAIN)
        pltpu.sync_copy(zf_hbm, acc_sh.at[sl])
        pltpu.sync_copy(zc_hbm, cnt_sh.at[sl])

      return carry

    lax.fori_loop(0, DRAIN_ITERS, zero_step, 0)

    for r in range(CHUNK):
      ones_v[r, :] = jnp.full((CHUNK,), 1.0, f32)

    plsc.subcore_barrier()

    def accumulate(feats_hbm, lab_hbm):
      def step(i, carry):
        base = sid * ROWS_PER_TILE + i * CHUNK
        pltpu.sync_copy(lab_hbm.at[pl.ds(base, CHUNK)], idx_v)
        pltpu.sync_copy(feats_hbm.at[pl.ds(base, CHUNK)], chunk_v)
        pltpu.sync_copy(chunk_v, acc_sh.at[idx_v], add=True)
        pltpu.sync_copy(ones_v, cnt_sh.at[idx_v], add=True)
        return carry

      lax.fori_loop(0, N_CHUNKS, step, 0)

    @pl.when(cid == 0)
    def _():
      accumulate(rgb_hbm, rgb_lab_hbm)

    @pl.when(cid == 1)
    def _():
      accumulate(ir_hbm, ir_lab_hbm)

    plsc.subcore_barrier()

    def drain(sum_hbm, cnt_hbm):
      def step(i, carry):
        k = sid + NS * i

        @pl.when(k < N_DRAIN)
        def _():
          sl = pl.ds(k * DRAIN, DRAIN)
          pltpu.sync_copy(acc_sh.at[sl], sum_hbm.at[sl])
          pltpu.sync_copy(cnt_sh.at[sl], cnt_hbm.at[sl])

        return carry

      lax.fori_loop(0, DRAIN_ITERS, step, 0)

    @pl.when(cid == 0)
    def _():
      drain(vis_sum_hbm, vis_cnt_hbm)

    @pl.when(cid == 1)
    def _():
      drain(ir_sum_hbm, ir_cnt_hbm)

  return body(rgb_feats, ir_feats, rgb_labels, ir_labels, zf, zc)


def _tc_finalize(vis_sums, ir_sums, vis_cnt, ir_cnt, vis_memory, ir_memory):
  BLK = 200
  f32 = jnp.float32

  def body(vs, irs, vc, ic, vm, im, out):
    for m, (s_ref, c_ref, mem_ref) in enumerate(
        ((vs, vc, vm), (irs, ic, im))):
      cnt = c_ref[:, 0:1]
      mem = mem_ref[...]
      mean = s_ref[...] / jnp.maximum(cnt, 1.0)
      upd = (1.0 - SIGMA) * mem + SIGMA * mean
      out[m] = jnp.where(cnt > 0.0, upd, mem)

  row_spec = pl.BlockSpec((BLK, FEAT), lambda i: (i, 0))
  cnt_spec = pl.BlockSpec((BLK, CHUNK), lambda i: (i, 0))
  return pl.pallas_call(
      body,
      grid=(NUM_CLASSES // BLK,),
      in_specs=[row_spec, row_spec, cnt_spec, cnt_spec, row_spec, row_spec],
      out_specs=pl.BlockSpec((2, BLK, FEAT), lambda i: (0, i, 0)),
      out_shape=jax.ShapeDtypeStruct((2, NUM_CLASSES, FEAT), f32),
  )(vis_sums, ir_sums, vis_cnt, ir_cnt, vis_memory, ir_memory)


def kernel(rgb_feats, ir_feats, rgb_labels, ir_labels, vis_memory, ir_memory):
  zf = jnp.zeros((DRAIN, FEAT), jnp.float32)
  zc = jnp.zeros((DRAIN, CHUNK), jnp.float32)
  vis_sums, ir_sums, vis_cnt, ir_cnt = _sc_segment_sums(
      rgb_feats, ir_feats,
      rgb_labels.astype(jnp.int32), ir_labels.astype(jnp.int32),
      zf, zc)
  return _tc_finalize(vis_sums, ir_sums, vis_cnt, ir_cnt,
                      vis_memory, ir_memory)


# speedup vs baseline: 2.7355x; 2.7355x over previous
import jax
import jax.numpy as jnp
from jax import lax
from jax.experimental import pallas as pl
from jax.experimental.pallas import tpu as pltpu

NUM_CLASSES = 1000
FEAT = 2048
BATCH = 16384
SIGMA = 0.2

BBLK = 256
NB = BATCH // BBLK


def _cma_update(rgb_feats, ir_feats, rgb_lab3d, ir_lab3d,
                vis_memory, ir_memory):
  f32 = jnp.float32

  def body(rgb_ref, ir_ref, rlab_ref, ilab_ref, vm_ref, im_ref, out_ref,
           vcnt, icnt):
    i = pl.program_id(0)

    @pl.when(i == 0)
    def _():
      out_ref[...] = jnp.zeros_like(out_ref)
      vcnt[...] = jnp.zeros_like(vcnt)
      icnt[...] = jnp.zeros_like(icnt)

    classes = lax.broadcasted_iota(jnp.int32, (BBLK, NUM_CLASSES), 1)
    for m, (cnt, lab_ref, f_ref) in enumerate(((vcnt, rlab_ref, rgb_ref),
                                               (icnt, ilab_ref, ir_ref))):
      onehot = (lab_ref[0, 0, :][:, None] == classes).astype(jnp.bfloat16)
      feats = f_ref[...].astype(jnp.bfloat16)
      out_ref[m] += lax.dot_general(
          onehot, feats, (((0,), (0,)), ((), ())),
          preferred_element_type=f32)
      cnt[...] += jnp.sum(onehot.astype(f32), axis=0, keepdims=True)

    @pl.when(i == NB - 1)
    def _():
      for m, (cnt, mem_ref) in enumerate(((vcnt, vm_ref), (icnt, im_ref))):
        c = cnt[0, :][:, None]
        mem = mem_ref[...]
        mean = out_ref[m] / jnp.maximum(c, 1.0)
        upd = (1.0 - SIGMA) * mem + SIGMA * mean
        out_ref[m] = jnp.where(c > 0.0, upd, mem)

  feat_spec = pl.BlockSpec((BBLK, FEAT), lambda i: (i, 0))
  lab_spec = pl.BlockSpec((1, 1, BBLK), lambda i: (i, 0, 0))
  mem_spec = pl.BlockSpec((NUM_CLASSES, FEAT), lambda i: (0, 0))
  return pl.pallas_call(
      body,
      grid=(NB,),
      in_specs=[feat_spec, feat_spec, lab_spec, lab_spec, mem_spec, mem_spec],
      out_specs=pl.BlockSpec((2, NUM_CLASSES, FEAT), lambda i: (0, 0, 0)),
      out_shape=jax.ShapeDtypeStruct((2, NUM_CLASSES, FEAT), f32),
      scratch_shapes=[
          pltpu.VMEM((1, NUM_CLASSES), f32),
          pltpu.VMEM((1, NUM_CLASSES), f32),
      ],
  )(rgb_feats, ir_feats, rgb_lab3d, ir_lab3d, vis_memory, ir_memory)


def kernel(rgb_feats, ir_feats, rgb_labels, ir_labels, vis_memory, ir_memory):
  rgb_lab3d = rgb_labels.astype(jnp.int32).reshape(NB, 1, BBLK)
  ir_lab3d = ir_labels.astype(jnp.int32).reshape(NB, 1, BBLK)
  return _cma_update(rgb_feats, ir_feats, rgb_lab3d, ir_lab3d,
                     vis_memory, ir_memory)


# speedup vs baseline: 2.8275x; 1.0336x over previous
import jax
import jax.numpy as jnp
from jax import lax
from jax.experimental import pallas as pl
from jax.experimental.pallas import tpu as pltpu

NUM_CLASSES = 1000
FEAT = 2048
BATCH = 16384
SIGMA = 0.2

BBLK = 1024
NB = BATCH // BBLK


def _segment_sums(rgb_feats, ir_feats, rgb_lab3d, ir_lab3d):
  f32 = jnp.float32

  def body(rgb_ref, ir_ref, rlab_ref, ilab_ref, sum_ref, cnt_ref):
    i = pl.program_id(0)

    @pl.when(i == 0)
    def _():
      sum_ref[...] = jnp.zeros_like(sum_ref)
      cnt_ref[...] = jnp.zeros_like(cnt_ref)

    classes = lax.broadcasted_iota(jnp.int32, (BBLK, NUM_CLASSES), 1)
    for m, (lab_ref, f_ref) in enumerate(((rlab_ref, rgb_ref),
                                          (ilab_ref, ir_ref))):
      onehot = (lab_ref[0, 0, :][:, None] == classes).astype(jnp.bfloat16)
      feats = f_ref[...].astype(jnp.bfloat16)
      sum_ref[m] += lax.dot_general(
          onehot, feats, (((0,), (0,)), ((), ())),
          preferred_element_type=f32)
      cnt_ref[m] += jnp.sum(onehot.astype(f32), axis=0, keepdims=True)

  feat_spec = pl.BlockSpec((BBLK, FEAT), lambda i: (i, 0))
  lab_spec = pl.BlockSpec((1, 1, BBLK), lambda i: (i, 0, 0))
  return pl.pallas_call(
      body,
      grid=(NB,),
      in_specs=[feat_spec, feat_spec, lab_spec, lab_spec],
      out_specs=[
          pl.BlockSpec((2, NUM_CLASSES, FEAT), lambda i: (0, 0, 0)),
          pl.BlockSpec((2, 1, NUM_CLASSES), lambda i: (0, 0, 0)),
      ],
      out_shape=[
          jax.ShapeDtypeStruct((2, NUM_CLASSES, FEAT), f32),
          jax.ShapeDtypeStruct((2, 1, NUM_CLASSES), f32),
      ],
  )(rgb_feats, ir_feats, rgb_lab3d, ir_lab3d)


def _tc_finalize(sums, counts, vis_memory, ir_memory):
  BLK = 200
  f32 = jnp.float32

  def body(s_ref, c_ref, vm_ref, im_ref, out_ref):
    for m, mem_ref in enumerate((vm_ref, im_ref)):
      c = c_ref[m, 0, 0, :][:, None]
      mem = mem_ref[...]
      mean = s_ref[m] / jnp.maximum(c, 1.0)
      upd = (1.0 - SIGMA) * mem + SIGMA * mean
      out_ref[m] = jnp.where(c > 0.0, upd, mem)

  row_spec = pl.BlockSpec((BLK, FEAT), lambda i: (i, 0))
  return pl.pallas_call(
      body,
      grid=(NUM_CLASSES // BLK,),
      in_specs=[
          pl.BlockSpec((2, BLK, FEAT), lambda i: (0, i, 0)),
          pl.BlockSpec((2, 1, 1, BLK), lambda i: (0, i, 0, 0)),
          row_spec, row_spec,
      ],
      out_specs=pl.BlockSpec((2, BLK, FEAT), lambda i: (0, i, 0)),
      out_shape=jax.ShapeDtypeStruct((2, NUM_CLASSES, FEAT), f32),
  )(sums, counts.reshape(2, NUM_CLASSES // BLK, 1, BLK),
    vis_memory, ir_memory)


def kernel(rgb_feats, ir_feats, rgb_labels, ir_labels, vis_memory, ir_memory):
  rgb_lab3d = rgb_labels.astype(jnp.int32).reshape(NB, 1, BBLK)
  ir_lab3d = ir_labels.astype(jnp.int32).reshape(NB, 1, BBLK)
  sums, counts = _segment_sums(rgb_feats, ir_feats, rgb_lab3d, ir_lab3d)
  return _tc_finalize(sums, counts, vis_memory, ir_memory)
